# baseline (speedup 1.0000x reference)
import functools

import jax
import jax.numpy as jnp
from jax import lax
from jax.experimental import pallas as pl
from jax.experimental.pallas import tpu as pltpu
from jax.experimental.pallas import tpu_sc as plsc

EMBEDDING_DIM = 64
_D = EMBEDDING_DIM
_L = 16

_info = plsc.get_sparse_core_info()
_NC, _NS = _info.num_cores, _info.num_subcores
_NW = _NC * _NS

_BW = 512
_DH = _D // 2


def _lookup_impl(n_t, n_b, table_t16, idx_t):
    assert n_b == _NW * _BW and n_t % 2 == 0 and _BW % _L == 0
    mesh = plsc.VectorSubcoreMesh(core_axis_name="c", subcore_axis_name="s")

    @functools.partial(
        pl.kernel,
        mesh=mesh,
        out_type=jax.ShapeDtypeStruct((n_t, _D, n_b), jnp.float32),
        scratch_types=[
            pltpu.VMEM((_D, _L), jnp.float32),
            pltpu.VMEM((_BW,), jnp.int32),
            pltpu.VMEM((_BW,), jnp.int32),
            pltpu.VMEM((_DH, _BW), jnp.float32),
            pltpu.VMEM((_DH, _BW), jnp.float32),
            pltpu.SemaphoreType.DMA,
            pltpu.SemaphoreType.DMA,
            pltpu.SemaphoreType.DMA,
            pltpu.SemaphoreType.DMA,
        ],
    )
    def k(table_hbm, idx_hbm, out_hbm, tbl_v, idxa, idxb, buf0, buf1,
          sem0, sem1, semia, semib):
        sid = lax.axis_index("s")
        wid = sid * _NC + lax.axis_index("c")
        b0 = pl.multiple_of(wid * _BW, _BW)
        pltpu.sync_copy(table_hbm, tbl_v)

        def half(t, h, idx_v, buf, sem):
            d0 = h * _DH

            @pl.when(t >= 1)
            def _wait_prev():
                pltpu.make_async_copy(
                    buf, out_hbm.at[t, pl.ds(d0, _DH), pl.ds(b0, _BW)],
                    sem).wait()

            cols = [tbl_v[d0 + j, :] for j in range(_DH)]

            @plsc.parallel_loop(0, _BW // _L, 1, unroll=8)
            def _bb(bb):
                o = pl.multiple_of(bb * _L, _L)
                idxv = idx_v[pl.ds(o, _L)]
                for j in range(_DH):
                    buf[j, pl.ds(o, _L)] = (
                        cols[j].at[idxv].get(mode="promise_in_bounds"))

            pltpu.async_copy(
                buf, out_hbm.at[t, pl.ds(d0, _DH), pl.ds(b0, _BW)], sem)

        def step(t, idx_v, semi):
            pltpu.make_async_copy(
                idx_hbm.at[t, pl.ds(b0, _BW)], idx_v, semi).wait()
            half(t, 0, idx_v, buf0, sem0)
            half(t, 1, idx_v, buf1, sem1)

        def body(tp, carry):
            t0 = 2 * tp
            pltpu.async_copy(idx_hbm.at[t0 + 1, pl.ds(b0, _BW)], idxb, semib)
            step(t0, idxa, semia)

            @pl.when(t0 + 2 < n_t)
            def _prefetch_next():
                pltpu.async_copy(
                    idx_hbm.at[t0 + 2, pl.ds(b0, _BW)], idxa, semia)

            step(t0 + 1, idxb, semib)
            return carry

        pltpu.async_copy(idx_hbm.at[0, pl.ds(b0, _BW)], idxa, semia)
        lax.fori_loop(0, n_t // 2, body, 0)
        pltpu.make_async_copy(
            buf0, out_hbm.at[0, pl.ds(0, _DH), pl.ds(b0, _BW)], sem0).wait()
        pltpu.make_async_copy(
            buf1, out_hbm.at[0, pl.ds(_DH, _DH), pl.ds(b0, _BW)], sem1).wait()

    return k(table_t16, idx_t)


def kernel(inputs, table):
    b, t = inputs.shape
    idx_t = inputs.T
    table_t16 = jnp.zeros((_D, _L), jnp.float32).at[:, :4].set(table.T)
    out_t = _lookup_impl(t, b, table_t16, idx_t)
    return jnp.transpose(out_t, (2, 0, 1))


# speedup vs baseline: 1.4179x; 1.4179x over previous
import functools

import jax
import jax.numpy as jnp
from jax import lax
from jax.experimental import pallas as pl
from jax.experimental.pallas import tpu as pltpu
from jax.experimental.pallas import tpu_sc as plsc

EMBEDDING_DIM = 64
_D = EMBEDDING_DIM
_L = 16

_info = plsc.get_sparse_core_info()
_NC, _NS = _info.num_cores, _info.num_subcores
_NW = _NC * _NS

_BW = 512
_DH = _D // 2


def _lookup_impl(n_t, n_b, table_t16, idx_t):
    assert n_b == _NW * _BW and n_t % 2 == 0 and _BW % _L == 0
    mesh = plsc.VectorSubcoreMesh(core_axis_name="c", subcore_axis_name="s")

    @functools.partial(
        pl.kernel,
        mesh=mesh,
        out_type=jax.ShapeDtypeStruct((n_t, _D, n_b), jnp.float32),
        scratch_types=[
            pltpu.VMEM((_D, _L), jnp.float32),
            pltpu.VMEM((_BW,), jnp.int32),
            pltpu.VMEM((_BW,), jnp.int32),
            pltpu.VMEM((_DH, _BW), jnp.float32),
            pltpu.VMEM((_DH, _BW), jnp.float32),
            pltpu.SemaphoreType.DMA,
            pltpu.SemaphoreType.DMA,
            pltpu.SemaphoreType.DMA,
            pltpu.SemaphoreType.DMA,
        ],
    )
    def k(table_hbm, idx_hbm, out_hbm, tbl_v, idxa, idxb, buf0, buf1,
          sem0, sem1, semia, semib):
        sid = lax.axis_index("s")
        wid = sid * _NC + lax.axis_index("c")
        b0 = pl.multiple_of(wid * _BW, _BW)
        pltpu.sync_copy(table_hbm, tbl_v)

        def half(t, h, idx_v, buf, sem):
            d0 = h * _DH

            @pl.when(t >= 1)
            def _wait_prev():
                pltpu.make_async_copy(
                    buf, out_hbm.at[t, pl.ds(d0, _DH), pl.ds(b0, _BW)],
                    sem).wait()

            cols = [tbl_v[d0 + j, :] for j in range(_DH)]

            @plsc.parallel_loop(0, _BW // _L, 1, unroll=4)
            def _bb(bb):
                o = pl.multiple_of(bb * _L, _L)
                idxv = idx_v[pl.ds(o, _L)]
                for j in range(_DH):
                    buf[j, pl.ds(o, _L)] = (
                        cols[j].at[idxv].get(mode="promise_in_bounds"))

            pltpu.async_copy(
                buf, out_hbm.at[t, pl.ds(d0, _DH), pl.ds(b0, _BW)], sem)

        def step(t, idx_v, semi):
            pltpu.make_async_copy(
                idx_hbm.at[t, pl.ds(b0, _BW)], idx_v, semi).wait()
            half(t, 0, idx_v, buf0, sem0)
            half(t, 1, idx_v, buf1, sem1)

        def body(tp, carry):
            t0 = 2 * tp
            pltpu.async_copy(idx_hbm.at[t0 + 1, pl.ds(b0, _BW)], idxb, semib)
            step(t0, idxa, semia)

            @pl.when(t0 + 2 < n_t)
            def _prefetch_next():
                pltpu.async_copy(
                    idx_hbm.at[t0 + 2, pl.ds(b0, _BW)], idxa, semia)

            step(t0 + 1, idxb, semib)
            return carry

        pltpu.async_copy(idx_hbm.at[0, pl.ds(b0, _BW)], idxa, semia)
        lax.fori_loop(0, n_t // 2, body, 0)
        pltpu.make_async_copy(
            buf0, out_hbm.at[0, pl.ds(0, _DH), pl.ds(b0, _BW)], sem0).wait()
        pltpu.make_async_copy(
            buf1, out_hbm.at[0, pl.ds(_DH, _DH), pl.ds(b0, _BW)], sem1).wait()

    return k(table_t16, idx_t)


def kernel(inputs, table):
    b, t = inputs.shape
    idx_t = inputs.T
    table_t16 = jnp.zeros((_D, _L), jnp.float32).at[:, :4].set(table.T)
    out_t = _lookup_impl(t, b, table_t16, idx_t)
    return jnp.transpose(out_t, (2, 0, 1))
